# baseline (speedup 1.0000x reference)
import functools

import jax
import jax.numpy as jnp
from jax import lax
from jax.experimental import pallas as pl
from jax.experimental.pallas import tpu as pltpu
from jax.experimental.pallas import tpu_sc as plsc

F_SP = 26
F_DN = 13
NF = F_SP + F_DN
D = 16
V = 100000
VPAD = 102400
EPS = 1e-5

NC = 2
NS = 16
NW = NC * NS
CHUNK = 128
NG = CHUNK // 16

QUART_V = VPAD // 4
NROW16 = F_SP * VPAD // 16

_SC_PARAMS = pltpu.CompilerParams(
    use_tc_tiling_on_sc=False, needs_layout_passes=False
)
_MESH = dict(core_axis_name="c", subcore_axis_name="s")


def _tc_rowsum(tables_t):
    def body(in_ref, out_ref):
        out_ref[...] = jnp.sum(in_ref[0], axis=0)

    return pl.pallas_call(
        body,
        grid=(F_SP, 4),
        in_specs=[pl.BlockSpec((1, D, QUART_V), lambda f, h: (f, 0, h))],
        out_specs=pl.BlockSpec((QUART_V,), lambda f, h: (f * 4 + h,)),
        out_shape=jax.ShapeDtypeStruct((F_SP * VPAD,), jnp.float32),
    )(tables_t)


def _sc_fused(inputs_flat, s16):
    b = inputs_flat.shape[0] // NF
    ngrp = b // CHUNK
    nchunk = ngrp // NW
    nrow = F_SP * CHUNK

    @functools.partial(
        pl.kernel,
        out_type=jax.ShapeDtypeStruct((ngrp, CHUNK), jnp.float32),
        mesh=plsc.VectorSubcoreMesh(**_MESH),
        scratch_types=[
            pltpu.VMEM((2, CHUNK * NF), jnp.float32),
            pltpu.VMEM((2, F_SP, CHUNK), jnp.int32),
            pltpu.VMEM((2, F_SP, CHUNK), jnp.int32),
            pltpu.VMEM((2 * nrow, D), jnp.float32),
            pltpu.VMEM((1, CHUNK), jnp.float32),
            pltpu.SemaphoreType.DMA,
            pltpu.SemaphoreType.DMA,
        ],
        compiler_params=_SC_PARAMS,
    )
    def k(in_hbm, s_hbm, out_hbm, raw_v, idx_v, off_v, rows_v, red_v, gsem, rsem):
        wid = lax.axis_index("c") * NS + lax.axis_index("s")
        lane = lax.iota(jnp.int32, 16)
        grp0 = wid * nchunk

        def fetch_raw(kc, slot):
            pltpu.async_copy(
                in_hbm.at[pl.ds((grp0 + kc) * CHUNK * NF, CHUNK * NF)],
                raw_v.at[slot],
                rsem,
            )

        def wait_raw(slot):
            pltpu.make_async_copy(
                in_hbm.at[pl.ds(0, CHUNK * NF)], raw_v.at[slot], rsem
            ).wait()

        def extract(slot):
            @pl.loop(0, F_SP)
            def _field(f):
                for g in range(NG):
                    pos = lane * NF + (g * 16 * NF) + f
                    ids = plsc.load_gather(raw_v.at[slot], [pos]).astype(jnp.int32)
                    p = ids + f * VPAD
                    idx_v[slot, f, pl.ds(g * 16, 16)] = p >> 4
                    off_v[slot, f, pl.ds(g * 16, 16)] = p & 15

        def fire_gathers(slot):
            @pl.loop(0, F_SP)
            def _field(f):
                pltpu.async_copy(
                    s_hbm.at[idx_v.at[slot, f]],
                    rows_v.at[pl.ds(slot * nrow + f * CHUNK, CHUNK), :],
                    gsem,
                )

        def drain_gathers():
            pltpu.make_async_copy(
                s_hbm.at[pl.ds(0, nrow), :],
                rows_v.at[pl.ds(0, nrow), :],
                gsem,
            ).wait()

        fetch_raw(0, 0)
        wait_raw(0)
        extract(0)
        fire_gathers(0)
        if nchunk > 1:
            fetch_raw(1, 1)

        @pl.loop(0, nchunk)
        def _chunk(kc):
            slot = lax.rem(kc, 2)

            @pl.when(kc + 1 < nchunk)
            def _():
                wait_raw(1 - slot)
                extract(1 - slot)

            drain_gathers()

            @pl.when(kc + 1 < nchunk)
            def _():
                fire_gathers(1 - slot)

            for g in range(NG):
                rbase = slot * nrow + lane + g * 16
                acc = plsc.load_gather(
                    rows_v, [rbase, off_v[slot, 0, pl.ds(g * 16, 16)]]
                )
                for f in range(1, F_SP):
                    acc = acc + plsc.load_gather(
                        rows_v,
                        [rbase + f * CHUNK, off_v[slot, f, pl.ds(g * 16, 16)]],
                    )
                red_v[0, pl.ds(g * 16, 16)] = acc

            @pl.when(kc + 2 < nchunk)
            def _():
                fetch_raw(kc + 2, slot)

            pltpu.sync_copy(red_v, out_hbm.at[pl.ds(grp0 + kc, 1), :])

    return k(inputs_flat, s16)


def _tc_combine(inputs, sp, gamma, beta, wt, bias):
    def body(in_ref, sp_ref, g_ref, b_ref, w_ref, bias_ref, out_ref):
        d = in_ref[:, F_SP:]
        mean = jnp.mean(d, axis=0, keepdims=True)
        c = d - mean
        var = jnp.mean(c * c, axis=0, keepdims=True)
        bn = c * lax.rsqrt(var + EPS) * g_ref[...][None, :] + b_ref[...][None, :]
        dense_logit = jnp.sum(bn * w_ref[...], axis=1, keepdims=True)
        out_ref[...] = sp_ref[...] + dense_logit + bias_ref[...][None, :]

    return pl.pallas_call(
        body,
        out_shape=jax.ShapeDtypeStruct((inputs.shape[0], 1), jnp.float32),
    )(inputs, sp, gamma, beta, wt, bias)


def kernel(inputs, tables, gamma, beta, W, bias):
    b = inputs.shape[0]
    s_flat = _tc_rowsum(jnp.transpose(tables, (0, 2, 1)))
    sp = _sc_fused(inputs.reshape(-1), s_flat.reshape(NROW16, D))
    wt = W.reshape(1, F_DN)
    return _tc_combine(inputs, sp.reshape(b, 1), gamma, beta, wt, bias)


# speedup vs baseline: 1.1870x; 1.1870x over previous
import functools

import jax
import jax.numpy as jnp
from jax import lax
from jax.experimental import pallas as pl
from jax.experimental.pallas import tpu as pltpu
from jax.experimental.pallas import tpu_sc as plsc

F_SP = 26
F_DN = 13
NF = F_SP + F_DN
D = 16
V = 100000
VPAD = 100352
EPS = 1e-5

NC = 2
NS = 16
NW = NC * NS
CHUNK = 128
NG = CHUNK // 16

HALF_V = VPAD // 2
NROW16 = F_SP * VPAD // 16

_SC_PARAMS = pltpu.CompilerParams(
    use_tc_tiling_on_sc=False, needs_layout_passes=False
)
_MESH = dict(core_axis_name="c", subcore_axis_name="s")


def _tc_rowsum(tables_t):
    def body(in_ref, out_ref):
        out_ref[...] = jnp.sum(in_ref[0], axis=0)

    return pl.pallas_call(
        body,
        grid=(F_SP, 2),
        in_specs=[pl.BlockSpec((1, D, HALF_V), lambda f, h: (f, 0, h))],
        out_specs=pl.BlockSpec((HALF_V,), lambda f, h: (f * 2 + h,)),
        out_shape=jax.ShapeDtypeStruct((F_SP * VPAD,), jnp.float32),
    )(tables_t)


def _sc_fused(inputs_flat, s16):
    b = inputs_flat.shape[0] // NF
    ngrp = b // CHUNK
    nchunk = ngrp // NW
    nrow = F_SP * CHUNK

    @functools.partial(
        pl.kernel,
        out_type=jax.ShapeDtypeStruct((ngrp, CHUNK), jnp.float32),
        mesh=plsc.VectorSubcoreMesh(**_MESH),
        scratch_types=[
            pltpu.VMEM((2, CHUNK * NF), jnp.float32),
            pltpu.VMEM((2, F_SP, CHUNK), jnp.int32),
            pltpu.VMEM((2, F_SP, CHUNK), jnp.int32),
            pltpu.VMEM((2 * nrow, D), jnp.float32),
            pltpu.VMEM((1, CHUNK), jnp.float32),
            pltpu.SemaphoreType.DMA,
            pltpu.SemaphoreType.DMA,
        ],
        compiler_params=_SC_PARAMS,
    )
    def k(in_hbm, s_hbm, out_hbm, raw_v, idx_v, off_v, rows_v, red_v, gsem, rsem):
        wid = lax.axis_index("c") * NS + lax.axis_index("s")
        lane = lax.iota(jnp.int32, 16)
        grp0 = wid * nchunk

        def fetch_raw(kc, slot):
            pltpu.async_copy(
                in_hbm.at[pl.ds((grp0 + kc) * CHUNK * NF, CHUNK * NF)],
                raw_v.at[slot],
                rsem,
            )

        def wait_raw(slot):
            pltpu.make_async_copy(
                in_hbm.at[pl.ds(0, CHUNK * NF)], raw_v.at[slot], rsem
            ).wait()

        def extract(slot):
            @pl.loop(0, F_SP)
            def _field(f):
                for g in range(NG):
                    pos = lane * NF + (g * 16 * NF) + f
                    ids = plsc.load_gather(raw_v.at[slot], [pos]).astype(jnp.int32)
                    p = ids + f * VPAD
                    idx_v[slot, f, pl.ds(g * 16, 16)] = p >> 4
                    off_v[slot, f, pl.ds(g * 16, 16)] = p & 15

        def fire_gathers(slot):
            @pl.loop(0, F_SP)
            def _field(f):
                pltpu.async_copy(
                    s_hbm.at[idx_v.at[slot, f]],
                    rows_v.at[pl.ds(slot * nrow + f * CHUNK, CHUNK), :],
                    gsem,
                )

        def drain_gathers():
            pltpu.make_async_copy(
                s_hbm.at[pl.ds(0, nrow), :],
                rows_v.at[pl.ds(0, nrow), :],
                gsem,
            ).wait()

        fetch_raw(0, 0)
        wait_raw(0)
        extract(0)
        fire_gathers(0)
        if nchunk > 1:
            fetch_raw(1, 1)

        @pl.loop(0, nchunk)
        def _chunk(kc):
            slot = lax.rem(kc, 2)

            @pl.when(kc + 1 < nchunk)
            def _():
                wait_raw(1 - slot)
                extract(1 - slot)

            drain_gathers()

            @pl.when(kc + 1 < nchunk)
            def _():
                fire_gathers(1 - slot)

            for g in range(NG):
                rbase = slot * nrow + lane + g * 16
                acc = plsc.load_gather(
                    rows_v, [rbase, off_v[slot, 0, pl.ds(g * 16, 16)]]
                )
                for f in range(1, F_SP):
                    acc = acc + plsc.load_gather(
                        rows_v,
                        [rbase + f * CHUNK, off_v[slot, f, pl.ds(g * 16, 16)]],
                    )
                red_v[0, pl.ds(g * 16, 16)] = acc

            @pl.when(kc + 2 < nchunk)
            def _():
                fetch_raw(kc + 2, slot)

            pltpu.sync_copy(red_v, out_hbm.at[pl.ds(grp0 + kc, 1), :])

    return k(inputs_flat, s16)


def _tc_combine(inputs, sp, gamma, beta, wt, bias):
    def body(in_ref, sp_ref, g_ref, b_ref, w_ref, bias_ref, out_ref):
        d = in_ref[:, F_SP:]
        mean = jnp.mean(d, axis=0, keepdims=True)
        c = d - mean
        var = jnp.mean(c * c, axis=0, keepdims=True)
        bn = c * lax.rsqrt(var + EPS) * g_ref[...][None, :] + b_ref[...][None, :]
        dense_logit = jnp.sum(bn * w_ref[...], axis=1, keepdims=True)
        out_ref[...] = sp_ref[...] + dense_logit + bias_ref[...][None, :]

    return pl.pallas_call(
        body,
        out_shape=jax.ShapeDtypeStruct((inputs.shape[0], 1), jnp.float32),
    )(inputs, sp, gamma, beta, wt, bias)


def kernel(inputs, tables, gamma, beta, W, bias):
    b = inputs.shape[0]
    s_flat = _tc_rowsum(jnp.transpose(tables, (0, 2, 1)))
    sp = _sc_fused(inputs.reshape(-1), s_flat.reshape(NROW16, D))
    wt = W.reshape(1, F_DN)
    return _tc_combine(inputs, sp.reshape(b, 1), gamma, beta, wt, bias)
